# baseline (device time: 22929 ns/iter reference)
import jax
import jax.numpy as jnp
from jax import lax
from jax.experimental import pallas as pl
from jax.experimental.pallas import tpu as pltpu

SIZES = (16, 16, 32, 48, 64, 64, 64, 64, 64, 48, 16, 16)
OFFS = tuple(sum(SIZES[:k]) for k in range(len(SIZES)))
C = len(SIZES)


def kernel(x, pi):
    _, m, n = x.shape
    half = m // 2
    assert sum(SIZES) == half

    x = pltpu.with_memory_space_constraint(x, pltpu.MemorySpace.HBM)
    pi = pltpu.with_memory_space_constraint(pi, pltpu.MemorySpace.SMEM)

    def body(x_ref, pi_ref, out_ref, sendx, xrecv, yrecv,
             sx, rx, sy, ry):
        my_x = lax.axis_index("x")
        my_y = lax.axis_index("y")
        dest_x = pi_ref[my_x]

        barrier_sem = pltpu.get_barrier_semaphore()
        pl.semaphore_signal(
            barrier_sem, inc=1, device_id=(dest_x, my_y),
            device_id_type=pl.DeviceIdType.MESH)
        pl.semaphore_signal(
            barrier_sem, inc=1, device_id=(my_x, 1 - my_y),
            device_id_type=pl.DeviceIdType.MESH)
        sendx[pl.ds(OFFS[0], SIZES[0]), :] = x_ref[
            0, pl.ds(OFFS[0], SIZES[0]), :].astype(jnp.bfloat16)
        pl.semaphore_wait(barrier_sem, 2)

        x_rdmas = []
        for k, (lo, sz) in enumerate(zip(OFFS, SIZES)):
            if k > 0:
                sendx[pl.ds(lo, sz), :] = x_ref[
                    0, pl.ds(lo, sz), :].astype(jnp.bfloat16)
            rdma = pltpu.make_async_remote_copy(
                src_ref=sendx.at[pl.ds(lo, sz), :],
                dst_ref=xrecv.at[pl.ds(lo, sz), :],
                send_sem=sx.at[k],
                recv_sem=rx.at[k],
                device_id=(dest_x, my_y),
                device_id_type=pl.DeviceIdType.MESH,
            )
            rdma.start()
            x_rdmas.append(rdma)

        half_off = my_y * half
        y_rdmas = []
        for k, (lo, sz) in enumerate(zip(OFFS, SIZES)):
            x_rdmas[k].wait_recv()
            fwd = pltpu.make_async_remote_copy(
                src_ref=xrecv.at[pl.ds(lo, sz), :],
                dst_ref=yrecv.at[pl.ds(lo, sz), :],
                send_sem=sy.at[k],
                recv_sem=ry.at[k],
                device_id=(my_x, 1 - my_y),
                device_id_type=pl.DeviceIdType.MESH,
            )
            fwd.start()
            y_rdmas.append(fwd)
            out_ref[0, pl.ds(half_off + lo, sz), :] = xrecv[pl.ds(lo, sz), :]

        other_off = (1 - my_y) * half
        for k, (lo, sz) in enumerate(zip(OFFS, SIZES)):
            y_rdmas[k].wait_recv()
            out_ref[0, pl.ds(other_off + lo, sz), :] = yrecv[pl.ds(lo, sz), :]

        for k in range(C):
            x_rdmas[k].wait_send()
            y_rdmas[k].wait_send()

    return pl.pallas_call(
        body,
        out_shape=jax.ShapeDtypeStruct((1, m, n), jnp.bfloat16),
        grid=(1,),
        in_specs=[
            pl.BlockSpec(
                (1, half, n), lambda i: (0, lax.axis_index("y"), 0)),
            pl.BlockSpec(memory_space=pltpu.SMEM),
        ],
        out_specs=pl.BlockSpec((1, m, n), lambda i: (0, 0, 0)),
        scratch_shapes=[
            pltpu.VMEM((half, n), jnp.bfloat16),
            pltpu.VMEM((half, n), jnp.bfloat16),
            pltpu.VMEM((half, n), jnp.bfloat16),
            pltpu.SemaphoreType.DMA((C,)),
            pltpu.SemaphoreType.DMA((C,)),
            pltpu.SemaphoreType.DMA((C,)),
            pltpu.SemaphoreType.DMA((C,)),
        ],
        compiler_params=pltpu.CompilerParams(collective_id=0),
    )(x, pi)


# device time: 22171 ns/iter; 1.0342x vs baseline; 1.0342x over previous
import jax
import jax.numpy as jnp
from jax import lax
from jax.experimental import pallas as pl
from jax.experimental.pallas import tpu as pltpu

SIZES = (16, 16, 32, 48, 64, 64, 64, 64, 64, 48, 16, 16)
OFFS = tuple(sum(SIZES[:k]) for k in range(len(SIZES)))
C = len(SIZES)


def kernel(x, pi):
    _, m, n = x.shape
    half = m // 2
    assert sum(SIZES) == half

    x = pltpu.with_memory_space_constraint(x, pltpu.MemorySpace.HBM)
    pi = pltpu.with_memory_space_constraint(pi, pltpu.MemorySpace.SMEM)

    NPIECE_A = 2
    a_rows = sum(SIZES[:NPIECE_A])

    def body(x_ref, pi_ref, out_ref, xstage, sendx, xrecv, yrecv,
             lsem, sx, rx, sy, ry):
        my_x = lax.axis_index("x")
        my_y = lax.axis_index("y")
        dest_x = pi_ref[my_x]
        half_off = my_y * half

        ld_a = pltpu.make_async_copy(
            x_ref.at[0, pl.ds(half_off, a_rows), :],
            xstage.at[pl.ds(0, a_rows), :], lsem.at[0])
        ld_a.start()
        ld_b = pltpu.make_async_copy(
            x_ref.at[0, pl.ds(half_off + a_rows, half - a_rows), :],
            xstage.at[pl.ds(a_rows, half - a_rows), :], lsem.at[1])
        ld_b.start()

        barrier_sem = pltpu.get_barrier_semaphore()
        pl.semaphore_signal(
            barrier_sem, inc=1, device_id=(dest_x, my_y),
            device_id_type=pl.DeviceIdType.MESH)
        pl.semaphore_signal(
            barrier_sem, inc=1, device_id=(my_x, 1 - my_y),
            device_id_type=pl.DeviceIdType.MESH)
        ld_a.wait()
        sendx[pl.ds(OFFS[0], SIZES[0]), :] = xstage[
            pl.ds(OFFS[0], SIZES[0]), :].astype(jnp.bfloat16)
        pl.semaphore_wait(barrier_sem, 2)

        x_rdmas = []
        for k, (lo, sz) in enumerate(zip(OFFS, SIZES)):
            if k == NPIECE_A:
                ld_b.wait()
            if k > 0:
                sendx[pl.ds(lo, sz), :] = xstage[
                    pl.ds(lo, sz), :].astype(jnp.bfloat16)
            rdma = pltpu.make_async_remote_copy(
                src_ref=sendx.at[pl.ds(lo, sz), :],
                dst_ref=xrecv.at[pl.ds(lo, sz), :],
                send_sem=sx.at[k],
                recv_sem=rx.at[k],
                device_id=(dest_x, my_y),
                device_id_type=pl.DeviceIdType.MESH,
            )
            rdma.start()
            x_rdmas.append(rdma)

        y_rdmas = []
        for k, (lo, sz) in enumerate(zip(OFFS, SIZES)):
            x_rdmas[k].wait_recv()
            fwd = pltpu.make_async_remote_copy(
                src_ref=xrecv.at[pl.ds(lo, sz), :],
                dst_ref=yrecv.at[pl.ds(lo, sz), :],
                send_sem=sy.at[k],
                recv_sem=ry.at[k],
                device_id=(my_x, 1 - my_y),
                device_id_type=pl.DeviceIdType.MESH,
            )
            fwd.start()
            y_rdmas.append(fwd)
            out_ref[0, pl.ds(half_off + lo, sz), :] = xrecv[pl.ds(lo, sz), :]

        other_off = (1 - my_y) * half
        for k, (lo, sz) in enumerate(zip(OFFS, SIZES)):
            y_rdmas[k].wait_recv()
            out_ref[0, pl.ds(other_off + lo, sz), :] = yrecv[pl.ds(lo, sz), :]

        for k in range(C):
            x_rdmas[k].wait_send()
            y_rdmas[k].wait_send()

    return pl.pallas_call(
        body,
        out_shape=jax.ShapeDtypeStruct((1, m, n), jnp.bfloat16),
        in_specs=[
            pl.BlockSpec(memory_space=pltpu.MemorySpace.HBM),
            pl.BlockSpec(memory_space=pltpu.SMEM),
        ],
        out_specs=pl.BlockSpec(memory_space=pltpu.VMEM),
        scratch_shapes=[
            pltpu.VMEM((half, n), jnp.float32),
            pltpu.VMEM((half, n), jnp.bfloat16),
            pltpu.VMEM((half, n), jnp.bfloat16),
            pltpu.VMEM((half, n), jnp.bfloat16),
            pltpu.SemaphoreType.DMA((2,)),
            pltpu.SemaphoreType.DMA((C,)),
            pltpu.SemaphoreType.DMA((C,)),
            pltpu.SemaphoreType.DMA((C,)),
            pltpu.SemaphoreType.DMA((C,)),
        ],
        compiler_params=pltpu.CompilerParams(collective_id=0),
    )(x, pi)
